# baseline (speedup 1.0000x reference)
import functools

import jax
import jax.numpy as jnp
from jax import lax
from jax.experimental import pallas as pl
from jax.experimental.pallas import tpu as pltpu
from jax.experimental.pallas import tpu_sc as plsc

K = 15
KPAD = 16
PD = 16
IN_DIM = 128
OUT_DIM = 128
N_NODES = 10000
N_PAD = 10112
E_EDGES = 160000
E_PAD = 163840
KP_EXTENT = 1.2

NC = 2
NS = 16
NW = NC * NS
L = 16
CHUNK = 128
GROUPS = CHUNK // L
EPW = E_PAD // NW
CHUNKS = EPW // CHUNK
ROWS_PER_TILE = N_PAD // NS


def _gather_body(src_hbm, dst_hbm, feat_hbm, px_hbm, py_hbm, pz_hbm,
                 fsrc_hbm, y_hbm,
                 sidx, didx, frows, yv, px, py, pz, sem):
    c = lax.axis_index("c")
    s = lax.axis_index("s")
    wid = s * NC + c
    pltpu.sync_copy(px_hbm, px)
    pltpu.sync_copy(py_hbm, py)
    pltpu.sync_copy(pz_hbm, pz)
    for r in range(CHUNK):
        yv[r] = jnp.zeros((L,), jnp.float32)

    lanes = lax.iota(jnp.int32, L)

    def body(i, carry):
        base = wid * EPW + i * CHUNK
        pltpu.sync_copy(src_hbm.at[pl.ds(base, CHUNK)], sidx)
        pltpu.sync_copy(dst_hbm.at[pl.ds(base, CHUNK)], didx)
        cp = pltpu.async_copy(feat_hbm.at[sidx], frows, sem)
        for g in range(GROUPS):
            ivs = sidx[pl.ds(g * L, L)]
            ivd = didx[pl.ds(g * L, L)]
            yx = plsc.load_gather(px, [ivs]) - plsc.load_gather(px, [ivd])
            yy = plsc.load_gather(py, [ivs]) - plsc.load_gather(py, [ivd])
            yz = plsc.load_gather(pz, [ivs]) - plsc.load_gather(pz, [ivd])
            rows = lanes + (g * L)
            plsc.store_scatter(yv, [rows, jnp.zeros((L,), jnp.int32)], yx)
            plsc.store_scatter(yv, [rows, jnp.ones((L,), jnp.int32)], yy)
            plsc.store_scatter(yv, [rows, jnp.full((L,), 2, jnp.int32)], yz)
        cp.wait()
        pltpu.sync_copy(frows, fsrc_hbm.at[pl.ds(base, CHUNK)])
        pltpu.sync_copy(yv, y_hbm.at[pl.ds(base, CHUNK)])
        return carry

    lax.fori_loop(0, CHUNKS, body, 0)


def _scatter_body(dst_hbm, msg_hbm, zeros_hbm, out_hbm, didx, mrows, acc, sem):
    c = lax.axis_index("c")
    s = lax.axis_index("s")
    wid = s * NC + c
    r0 = s * ROWS_PER_TILE
    pltpu.sync_copy(zeros_hbm.at[pl.ds(r0, ROWS_PER_TILE)],
                    acc.at[pl.ds(r0, ROWS_PER_TILE)])
    plsc.subcore_barrier()

    def body(i, carry):
        base = wid * EPW + i * CHUNK
        pltpu.sync_copy(dst_hbm.at[pl.ds(base, CHUNK)], didx)
        pltpu.sync_copy(msg_hbm.at[pl.ds(base, CHUNK)], mrows)
        pltpu.sync_copy(mrows, acc.at[didx], add=True)
        return carry

    lax.fori_loop(0, CHUNKS, body, 0)
    plsc.subcore_barrier()
    pltpu.sync_copy(acc.at[pl.ds(r0, ROWS_PER_TILE)],
                    out_hbm.at[c, pl.ds(r0, ROWS_PER_TILE)])


BB = 512


def _msg_body(fsrc_ref, y_ref, kpt_ref, w_ref, msg_ref):
    y = y_ref[...]
    kpt = kpt_ref[...]
    cross = jnp.dot(y, kpt, preferred_element_type=jnp.float32)
    yn2 = jnp.sum(y * y, axis=1, keepdims=True)
    kn2 = jnp.sum(kpt * kpt, axis=0, keepdims=True)
    d2 = jnp.maximum(yn2 + kn2 - 2.0 * cross, 0.0) + 1e-12
    h = jnp.maximum(1.0 - jnp.sqrt(d2) * (1.0 / KP_EXTENT), 0.0)
    f = fsrc_ref[...]
    hf = (h[:, :, None] * f[:, None, :]).reshape(BB, KPAD * IN_DIM)
    msg_ref[...] = jnp.dot(hf.astype(jnp.bfloat16), w_ref[...],
                           preferred_element_type=jnp.float32)


def _add_body(a_ref, b_ref, o_ref):
    o_ref[...] = a_ref[...] + b_ref[...]


@jax.jit
def kernel(feat, pos, edge_index, weights, kernel_points):
    src = edge_index[0]
    dst = edge_index[1]
    epad = E_PAD - E_EDGES
    src_p = jnp.concatenate([src, jnp.zeros((epad,), jnp.int32)])
    dst_p = jnp.concatenate([dst, jnp.full((epad,), N_NODES, jnp.int32)])
    px, py, pz = pos[:, 0], pos[:, 1], pos[:, 2]
    kpt = jnp.pad(kernel_points, ((0, KPAD - K), (0, PD - kernel_points.shape[1]))).T
    w_flat = jnp.pad(weights, ((0, KPAD - K), (0, 0), (0, 0))).reshape(
        KPAD * IN_DIM, OUT_DIM).astype(jnp.bfloat16)

    mesh = plsc.VectorSubcoreMesh(core_axis_name="c", subcore_axis_name="s")

    gather_fn = pl.kernel(
        _gather_body,
        out_type=[
            jax.ShapeDtypeStruct((E_PAD, IN_DIM), jnp.float32),
            jax.ShapeDtypeStruct((E_PAD, PD), jnp.float32),
        ],
        mesh=mesh,
        compiler_params=pltpu.CompilerParams(needs_layout_passes=False),
        scratch_types=[
            pltpu.VMEM((CHUNK,), jnp.int32),
            pltpu.VMEM((CHUNK,), jnp.int32),
            pltpu.VMEM((CHUNK, IN_DIM), jnp.float32),
            pltpu.VMEM((CHUNK, PD), jnp.float32),
            pltpu.VMEM((N_NODES,), jnp.float32),
            pltpu.VMEM((N_NODES,), jnp.float32),
            pltpu.VMEM((N_NODES,), jnp.float32),
            pltpu.SemaphoreType.DMA,
        ],
    )
    fsrc, yarr = gather_fn(src_p, dst_p, feat, px, py, pz)

    msg = pl.pallas_call(
        _msg_body,
        grid=(E_PAD // BB,),
        in_specs=[
            pl.BlockSpec((BB, IN_DIM), lambda i: (i, 0)),
            pl.BlockSpec((BB, PD), lambda i: (i, 0)),
            pl.BlockSpec((PD, KPAD), lambda i: (0, 0)),
            pl.BlockSpec((KPAD * IN_DIM, OUT_DIM), lambda i: (0, 0)),
        ],
        out_specs=pl.BlockSpec((BB, OUT_DIM), lambda i: (i, 0)),
        out_shape=jax.ShapeDtypeStruct((E_PAD, OUT_DIM), jnp.float32),
    )(fsrc, yarr, kpt, w_flat)

    scatter_fn = pl.kernel(
        _scatter_body,
        out_type=jax.ShapeDtypeStruct((NC, N_PAD, OUT_DIM), jnp.float32),
        mesh=mesh,
        scratch_types=[
            pltpu.VMEM((CHUNK,), jnp.int32),
            pltpu.VMEM((CHUNK, OUT_DIM), jnp.float32),
            pltpu.VMEM_SHARED((N_PAD, OUT_DIM), jnp.float32),
            pltpu.SemaphoreType.DMA,
        ],
    )
    zeros_hbm = jnp.zeros((N_PAD, OUT_DIM), jnp.float32)
    partials = scatter_fn(dst_p, msg, zeros_hbm)

    out = pl.pallas_call(
        _add_body,
        grid=(10,),
        in_specs=[
            pl.BlockSpec((1000, OUT_DIM), lambda i: (i, 0)),
            pl.BlockSpec((1000, OUT_DIM), lambda i: (i, 0)),
        ],
        out_specs=pl.BlockSpec((1000, OUT_DIM), lambda i: (i, 0)),
        out_shape=jax.ShapeDtypeStruct((N_NODES, OUT_DIM), jnp.float32),
    )(partials[0, :N_NODES], partials[1, :N_NODES])
    return out


# speedup vs baseline: 1.1911x; 1.1911x over previous
import functools

import jax
import jax.numpy as jnp
from jax import lax
from jax.experimental import pallas as pl
from jax.experimental.pallas import tpu as pltpu
from jax.experimental.pallas import tpu_sc as plsc

K = 15
KPAD = 16
PD = 16
IN_DIM = 128
OUT_DIM = 128
N_NODES = 10000
N_PAD = 10112
E_EDGES = 160000
E_PAD = 163840
KP_EXTENT = 1.2

NC = 2
NS = 16
NW = NC * NS
L = 16
CHUNK = 128
GROUPS = CHUNK // L
EPW = E_PAD // NW
CHUNKS = EPW // CHUNK
ROWS_PER_TILE = N_PAD // NS


def _gather_body(src_hbm, dst_hbm, feat_hbm, px_hbm, py_hbm, pz_hbm,
                 fsrc_hbm, y_hbm,
                 sidx, didx, frows, yv, px, py, pz, sem):
    c = lax.axis_index("c")
    s = lax.axis_index("s")
    wid = s * NC + c
    pltpu.sync_copy(px_hbm, px)
    pltpu.sync_copy(py_hbm, py)
    pltpu.sync_copy(pz_hbm, pz)
    for r in range(CHUNK):
        yv[r] = jnp.zeros((L,), jnp.float32)

    lanes = lax.iota(jnp.int32, L)

    def body(i, carry):
        base = wid * EPW + i * CHUNK
        pltpu.sync_copy(src_hbm.at[pl.ds(base, CHUNK)], sidx)
        pltpu.sync_copy(dst_hbm.at[pl.ds(base, CHUNK)], didx)
        cp = pltpu.async_copy(feat_hbm.at[sidx], frows, sem)
        for g in range(GROUPS):
            ivs = sidx[pl.ds(g * L, L)]
            ivd = didx[pl.ds(g * L, L)]
            yx = plsc.load_gather(px, [ivs]) - plsc.load_gather(px, [ivd])
            yy = plsc.load_gather(py, [ivs]) - plsc.load_gather(py, [ivd])
            yz = plsc.load_gather(pz, [ivs]) - plsc.load_gather(pz, [ivd])
            rows = lanes + (g * L)
            plsc.store_scatter(yv, [rows, jnp.zeros((L,), jnp.int32)], yx)
            plsc.store_scatter(yv, [rows, jnp.ones((L,), jnp.int32)], yy)
            plsc.store_scatter(yv, [rows, jnp.full((L,), 2, jnp.int32)], yz)
        cp.wait()
        pltpu.sync_copy(frows, fsrc_hbm.at[pl.ds(base, CHUNK)])
        pltpu.sync_copy(yv, y_hbm.at[pl.ds(base, CHUNK)])
        return carry

    lax.fori_loop(0, CHUNKS, body, 0)


def _scatter_body(dst_hbm, msg_hbm, zeros_hbm, out_hbm, didx, mrows, acc, sem):
    c = lax.axis_index("c")
    s = lax.axis_index("s")
    wid = s * NC + c
    r0 = s * ROWS_PER_TILE
    pltpu.sync_copy(zeros_hbm.at[pl.ds(r0, ROWS_PER_TILE)],
                    acc.at[pl.ds(r0, ROWS_PER_TILE)])
    plsc.subcore_barrier()

    def body(i, carry):
        base = wid * EPW + i * CHUNK
        pltpu.sync_copy(dst_hbm.at[pl.ds(base, CHUNK)], didx)
        pltpu.sync_copy(msg_hbm.at[pl.ds(base, CHUNK)], mrows)
        pltpu.sync_copy(mrows, acc.at[didx], add=True)
        return carry

    lax.fori_loop(0, CHUNKS, body, 0)
    plsc.subcore_barrier()
    pltpu.sync_copy(acc.at[pl.ds(r0, ROWS_PER_TILE)],
                    out_hbm.at[c, pl.ds(r0, ROWS_PER_TILE)])


BB = 1024


def _msg_body(fsrc_ref, y_ref, kpt_ref, wall_ref, sel_ref, msg_ref):
    y = y_ref[...]
    kpt = kpt_ref[...]
    cross = jnp.dot(y, kpt, preferred_element_type=jnp.float32)
    yn2 = jnp.sum(y * y, axis=1, keepdims=True)
    kn2 = jnp.sum(kpt * kpt, axis=0, keepdims=True)
    d2 = jnp.maximum(yn2 + kn2 - 2.0 * cross, 0.0) + 1e-12
    h = jnp.maximum(1.0 - jnp.sqrt(d2) * (1.0 / KP_EXTENT), 0.0)
    f16 = fsrc_ref[...].astype(jnp.bfloat16)
    t = jnp.dot(f16, wall_ref[...], preferred_element_type=jnp.float32)
    hb = jnp.dot(h.astype(jnp.bfloat16), sel_ref[...],
                 preferred_element_type=jnp.float32)
    acc = hb[:, 0:OUT_DIM] * t[:, 0:OUT_DIM]
    for k in range(1, KPAD):
        sl = slice(k * OUT_DIM, (k + 1) * OUT_DIM)
        acc = acc + hb[:, sl] * t[:, sl]
    msg_ref[...] = acc


def _add_body(a_ref, b_ref, o_ref):
    o_ref[...] = a_ref[...] + b_ref[...]


@jax.jit
def kernel(feat, pos, edge_index, weights, kernel_points):
    src = edge_index[0]
    dst = edge_index[1]
    epad = E_PAD - E_EDGES
    src_p = jnp.concatenate([src, jnp.zeros((epad,), jnp.int32)])
    dst_p = jnp.concatenate([dst, jnp.full((epad,), N_NODES, jnp.int32)])
    px, py, pz = pos[:, 0], pos[:, 1], pos[:, 2]
    kpt = jnp.pad(kernel_points, ((0, KPAD - K), (0, PD - kernel_points.shape[1]))).T
    w_all = jnp.transpose(
        jnp.pad(weights, ((0, KPAD - K), (0, 0), (0, 0))), (1, 0, 2)
    ).reshape(IN_DIM, KPAD * OUT_DIM).astype(jnp.bfloat16)
    sel = jnp.repeat(jnp.eye(KPAD, dtype=jnp.float32), OUT_DIM,
                     axis=1).astype(jnp.bfloat16)

    mesh = plsc.VectorSubcoreMesh(core_axis_name="c", subcore_axis_name="s")

    gather_fn = pl.kernel(
        _gather_body,
        out_type=[
            jax.ShapeDtypeStruct((E_PAD, IN_DIM), jnp.float32),
            jax.ShapeDtypeStruct((E_PAD, PD), jnp.float32),
        ],
        mesh=mesh,
        compiler_params=pltpu.CompilerParams(needs_layout_passes=False),
        scratch_types=[
            pltpu.VMEM((CHUNK,), jnp.int32),
            pltpu.VMEM((CHUNK,), jnp.int32),
            pltpu.VMEM((CHUNK, IN_DIM), jnp.float32),
            pltpu.VMEM((CHUNK, PD), jnp.float32),
            pltpu.VMEM((N_NODES,), jnp.float32),
            pltpu.VMEM((N_NODES,), jnp.float32),
            pltpu.VMEM((N_NODES,), jnp.float32),
            pltpu.SemaphoreType.DMA,
        ],
    )
    fsrc, yarr = gather_fn(src_p, dst_p, feat, px, py, pz)

    msg = pl.pallas_call(
        _msg_body,
        grid=(E_PAD // BB,),
        in_specs=[
            pl.BlockSpec((BB, IN_DIM), lambda i: (i, 0)),
            pl.BlockSpec((BB, PD), lambda i: (i, 0)),
            pl.BlockSpec((PD, KPAD), lambda i: (0, 0)),
            pl.BlockSpec((IN_DIM, KPAD * OUT_DIM), lambda i: (0, 0)),
            pl.BlockSpec((KPAD, KPAD * OUT_DIM), lambda i: (0, 0)),
        ],
        out_specs=pl.BlockSpec((BB, OUT_DIM), lambda i: (i, 0)),
        out_shape=jax.ShapeDtypeStruct((E_PAD, OUT_DIM), jnp.float32),
    )(fsrc, yarr, kpt, w_all, sel)

    scatter_fn = pl.kernel(
        _scatter_body,
        out_type=jax.ShapeDtypeStruct((NC, N_PAD, OUT_DIM), jnp.float32),
        mesh=mesh,
        scratch_types=[
            pltpu.VMEM((CHUNK,), jnp.int32),
            pltpu.VMEM((CHUNK, OUT_DIM), jnp.float32),
            pltpu.VMEM_SHARED((N_PAD, OUT_DIM), jnp.float32),
            pltpu.SemaphoreType.DMA,
        ],
    )
    zeros_hbm = jnp.zeros((N_PAD, OUT_DIM), jnp.float32)
    partials = scatter_fn(dst_p, msg, zeros_hbm)

    out = pl.pallas_call(
        _add_body,
        grid=(10,),
        in_specs=[
            pl.BlockSpec((1000, OUT_DIM), lambda i: (i, 0)),
            pl.BlockSpec((1000, OUT_DIM), lambda i: (i, 0)),
        ],
        out_specs=pl.BlockSpec((1000, OUT_DIM), lambda i: (i, 0)),
        out_shape=jax.ShapeDtypeStruct((N_NODES, OUT_DIM), jnp.float32),
    )(partials[0, :N_NODES], partials[1, :N_NODES])
    return out


# speedup vs baseline: 1.3546x; 1.1372x over previous
import functools

import jax
import jax.numpy as jnp
from jax import lax
from jax.experimental import pallas as pl
from jax.experimental.pallas import tpu as pltpu
from jax.experimental.pallas import tpu_sc as plsc

K = 15
KPAD = 16
PD = 8
IN_DIM = 128
OUT_DIM = 128
N_NODES = 10000
N_PAD = 10112
E_EDGES = 160000
E_PAD = 163840
KP_EXTENT = 1.2

NC = 2
NS = 16
NW = NC * NS
L = 16
CHUNK = 128
GROUPS = CHUNK // L
EPW = E_PAD // NW
CHUNKS = EPW // CHUNK
ROWS_PER_TILE = N_PAD // NS


STEP = 160
HALF = STEP // 2
NSTEP = EPW // STEP
HGROUPS = HALF // L


def _gather_body(src_hbm, dst_hbm, feat_hbm, posf_hbm,
                 fsrc_hbm, y_hbm,
                 sidx, didx, frows, yv, pf,
                 si0, si1, sg0, sg1, sw0, sw1):
    c = lax.axis_index("c")
    s = lax.axis_index("s")
    wid = s * NC + c
    base_w = wid * EPW
    si = (si0, si1)
    sg = (sg0, sg1)
    sw = (sw0, sw1)
    pltpu.sync_copy(posf_hbm, pf)
    lanes = lax.iota(jnp.int32, L)

    def issue_idx(t, b):
        base = base_w + t * STEP
        for j in range(2):
            pltpu.async_copy(src_hbm.at[pl.ds(base + j * HALF, HALF)],
                             sidx.at[b, j], si[b])
            pltpu.async_copy(dst_hbm.at[pl.ds(base + j * HALF, HALF)],
                             didx.at[b, j], si[b])

    def wait_idx(b):
        for j in range(2):
            pltpu.make_async_copy(src_hbm.at[pl.ds(0, HALF)], sidx.at[b, j],
                                  si[b]).wait()
            pltpu.make_async_copy(dst_hbm.at[pl.ds(0, HALF)], didx.at[b, j],
                                  si[b]).wait()

    def issue_gather(b):
        for j in range(2):
            pltpu.async_copy(
                feat_hbm.at[sidx.at[b, j]],
                frows.at[b, pl.ds(j * HALF, HALF)], sg[b])

    def wait_gather(b):
        pltpu.make_async_copy(feat_hbm.at[pl.ds(0, STEP)], frows.at[b],
                              sg[b]).wait()

    def issue_wout(t, b):
        base = base_w + t * STEP
        pltpu.async_copy(frows.at[b], fsrc_hbm.at[pl.ds(base, STEP)], sw[b])
        pltpu.async_copy(yv.at[b], y_hbm.at[pl.ds(base, STEP)], sw[b])

    def wait_wout(b):
        pltpu.make_async_copy(frows.at[b], fsrc_hbm.at[pl.ds(0, STEP)],
                              sw[b]).wait()
        pltpu.make_async_copy(yv.at[b], y_hbm.at[pl.ds(0, STEP)], sw[b]).wait()

    def compute_y(b):
        for j in range(2):
            for g in range(HGROUPS):
                ivs = sidx[b, j, pl.ds(g * L, L)]
                ivd = didx[b, j, pl.ds(g * L, L)]
                yx = plsc.load_gather(pf, [ivs]) - plsc.load_gather(pf, [ivd])
                ivs = ivs + N_NODES
                ivd = ivd + N_NODES
                yy = plsc.load_gather(pf, [ivs]) - plsc.load_gather(pf, [ivd])
                ivs = ivs + N_NODES
                ivd = ivd + N_NODES
                yz = plsc.load_gather(pf, [ivs]) - plsc.load_gather(pf, [ivd])
                rows = lanes + (j * HALF + g * L)
                plsc.store_scatter(yv.at[b],
                                   [rows, jnp.zeros((L,), jnp.int32)], yx)
                plsc.store_scatter(yv.at[b],
                                   [rows, jnp.ones((L,), jnp.int32)], yy)
                plsc.store_scatter(yv.at[b],
                                   [rows, jnp.full((L,), 2, jnp.int32)], yz)

    issue_idx(0, 0)
    issue_idx(1, 1)

    def body(o, carry):
        for b in range(2):
            t = 2 * o + b
            wait_idx(b)

            @pl.when(o >= 1)
            def _():
                wait_wout(b)

            issue_gather(b)
            compute_y(b)
            wait_gather(b)
            issue_wout(t, b)

            @pl.when(t + 2 < NSTEP)
            def _():
                issue_idx(t + 2, b)
        return carry

    lax.fori_loop(0, NSTEP // 2, body, 0)
    wait_wout(0)
    wait_wout(1)


def _scatter_body(dst_hbm, msg_hbm, zeros_hbm, out_hbm, didx, mrows, acc,
                  sl0, sl1, ss0, ss1):
    c = lax.axis_index("c")
    s = lax.axis_index("s")
    wid = s * NC + c
    base_w = wid * EPW
    sl = (sl0, sl1)
    ss = (ss0, ss1)
    r0 = s * ROWS_PER_TILE
    pltpu.sync_copy(zeros_hbm.at[pl.ds(r0, ROWS_PER_TILE)],
                    acc.at[pl.ds(r0, ROWS_PER_TILE)])
    plsc.subcore_barrier()

    def issue_load(t, b):
        base = base_w + t * CHUNK
        pltpu.async_copy(dst_hbm.at[pl.ds(base, CHUNK)], didx.at[b], sl[b])
        pltpu.async_copy(msg_hbm.at[pl.ds(base, CHUNK)], mrows.at[b], sl[b])

    def wait_load(b):
        pltpu.make_async_copy(dst_hbm.at[pl.ds(0, CHUNK)], didx.at[b],
                              sl[b]).wait()
        pltpu.make_async_copy(msg_hbm.at[pl.ds(0, CHUNK)], mrows.at[b],
                              sl[b]).wait()

    def wait_scatter(b):
        pltpu.make_async_copy(msg_hbm.at[pl.ds(0, CHUNK)], mrows.at[b],
                              ss[b]).wait()

    issue_load(0, 0)

    def body(o, carry):
        for b in range(2):
            t = 2 * o + b
            wait_load(b)
            pltpu.async_copy(mrows.at[b], acc.at[didx.at[b]], ss[b], add=True)

            @pl.when(t >= 1)
            def _():
                wait_scatter(1 - b)

            @pl.when(t + 1 < CHUNKS)
            def _():
                issue_load(t + 1, 1 - b)
        return carry

    lax.fori_loop(0, CHUNKS // 2, body, 0)
    wait_scatter(1)
    plsc.subcore_barrier()
    pltpu.sync_copy(acc.at[pl.ds(r0, ROWS_PER_TILE)],
                    out_hbm.at[c, pl.ds(r0, ROWS_PER_TILE)])


BB = 1024


def _msg_body(fsrc_ref, y_ref, kpt_ref, wall_ref, sel_ref, msg_ref):
    col = lax.broadcasted_iota(jnp.int32, (BB, PD), 1)
    y = jnp.where(col < 3, y_ref[...], 0.0)
    kpt = kpt_ref[...]
    cross = jnp.dot(y, kpt, preferred_element_type=jnp.float32)
    yn2 = jnp.sum(y * y, axis=1, keepdims=True)
    kn2 = jnp.sum(kpt * kpt, axis=0, keepdims=True)
    d2 = jnp.maximum(yn2 + kn2 - 2.0 * cross, 0.0) + 1e-12
    h = jnp.maximum(1.0 - jnp.sqrt(d2) * (1.0 / KP_EXTENT), 0.0)
    f16 = fsrc_ref[...].astype(jnp.bfloat16)
    t = jnp.dot(f16, wall_ref[...], preferred_element_type=jnp.float32)
    hb = jnp.dot(h.astype(jnp.bfloat16), sel_ref[...],
                 preferred_element_type=jnp.float32)
    acc = hb[:, 0:OUT_DIM] * t[:, 0:OUT_DIM]
    for k in range(1, KPAD):
        sl = slice(k * OUT_DIM, (k + 1) * OUT_DIM)
        acc = acc + hb[:, sl] * t[:, sl]
    msg_ref[...] = acc


def _add_body(a_ref, b_ref, o_ref):
    o_ref[...] = a_ref[...] + b_ref[...]


@jax.jit
def kernel(feat, pos, edge_index, weights, kernel_points):
    src = edge_index[0]
    dst = edge_index[1]
    epad = E_PAD - E_EDGES
    src_p = jnp.concatenate([src, jnp.zeros((epad,), jnp.int32)])
    dst_p = jnp.concatenate([dst, jnp.full((epad,), N_NODES, jnp.int32)])
    posf = jnp.concatenate([pos[:, 0], pos[:, 1], pos[:, 2]])
    kpt = jnp.pad(kernel_points, ((0, KPAD - K), (0, PD - kernel_points.shape[1]))).T
    w_all = jnp.transpose(
        jnp.pad(weights, ((0, KPAD - K), (0, 0), (0, 0))), (1, 0, 2)
    ).reshape(IN_DIM, KPAD * OUT_DIM).astype(jnp.bfloat16)
    sel = jnp.repeat(jnp.eye(KPAD, dtype=jnp.float32), OUT_DIM,
                     axis=1).astype(jnp.bfloat16)

    mesh = plsc.VectorSubcoreMesh(core_axis_name="c", subcore_axis_name="s")

    gather_fn = pl.kernel(
        _gather_body,
        out_type=[
            jax.ShapeDtypeStruct((E_PAD, IN_DIM), jnp.float32),
            jax.ShapeDtypeStruct((E_PAD, PD), jnp.float32),
        ],
        mesh=mesh,
        compiler_params=pltpu.CompilerParams(needs_layout_passes=False),
        scratch_types=[
            pltpu.VMEM((2, 2, HALF), jnp.int32),
            pltpu.VMEM((2, 2, HALF), jnp.int32),
            pltpu.VMEM((2, STEP, IN_DIM), jnp.float32),
            pltpu.VMEM((2, STEP, PD), jnp.float32),
            pltpu.VMEM((3 * N_NODES,), jnp.float32),
            pltpu.SemaphoreType.DMA,
            pltpu.SemaphoreType.DMA,
            pltpu.SemaphoreType.DMA,
            pltpu.SemaphoreType.DMA,
            pltpu.SemaphoreType.DMA,
            pltpu.SemaphoreType.DMA,
        ],
    )
    fsrc, yarr = gather_fn(src_p, dst_p, feat, posf)

    msg = pl.pallas_call(
        _msg_body,
        grid=(E_PAD // BB,),
        in_specs=[
            pl.BlockSpec((BB, IN_DIM), lambda i: (i, 0)),
            pl.BlockSpec((BB, PD), lambda i: (i, 0)),
            pl.BlockSpec((PD, KPAD), lambda i: (0, 0)),
            pl.BlockSpec((IN_DIM, KPAD * OUT_DIM), lambda i: (0, 0)),
            pl.BlockSpec((KPAD, KPAD * OUT_DIM), lambda i: (0, 0)),
        ],
        out_specs=pl.BlockSpec((BB, OUT_DIM), lambda i: (i, 0)),
        out_shape=jax.ShapeDtypeStruct((E_PAD, OUT_DIM), jnp.float32),
    )(fsrc, yarr, kpt, w_all, sel)

    scatter_fn = pl.kernel(
        _scatter_body,
        out_type=jax.ShapeDtypeStruct((NC, N_PAD, OUT_DIM), jnp.float32),
        mesh=mesh,
        scratch_types=[
            pltpu.VMEM((2, CHUNK), jnp.int32),
            pltpu.VMEM((2, CHUNK, OUT_DIM), jnp.float32),
            pltpu.VMEM_SHARED((N_PAD, OUT_DIM), jnp.float32),
            pltpu.SemaphoreType.DMA,
            pltpu.SemaphoreType.DMA,
            pltpu.SemaphoreType.DMA,
            pltpu.SemaphoreType.DMA,
        ],
    )
    zeros_hbm = jnp.zeros((N_PAD, OUT_DIM), jnp.float32)
    partials = scatter_fn(dst_p, msg, zeros_hbm)

    out = pl.pallas_call(
        _add_body,
        grid=(10,),
        in_specs=[
            pl.BlockSpec((1000, OUT_DIM), lambda i: (i, 0)),
            pl.BlockSpec((1000, OUT_DIM), lambda i: (i, 0)),
        ],
        out_specs=pl.BlockSpec((1000, OUT_DIM), lambda i: (i, 0)),
        out_shape=jax.ShapeDtypeStruct((N_NODES, OUT_DIM), jnp.float32),
    )(partials[0, :N_NODES], partials[1, :N_NODES])
    return out
